# baseline (speedup 1.0000x reference)
import functools
import jax
import jax.numpy as jnp
from jax import lax
from jax.experimental import pallas as pl
from jax.experimental.pallas import tpu as pltpu
from jax.experimental.pallas import tpu_sc as plsc

NEG = 0.1
NL, NP, NF, NR, D, DX = 2048, 8192, 128, 512, 256, 32
DS = D + DX
FD = 2 * DS
L_INTRA = 2


def _lrelu(x):
    return jnp.where(x >= 0, x, NEG * x)


def _attn_body(q_ref, k_ref, v_ref, cq_ref, ck_ref, o_ref,
               acc_ref, m_ref, l_ref, *, scale):
    j = pl.program_id(1)
    nj = pl.num_programs(1)

    @pl.when(j == 0)
    def _():
        m_ref[...] = jnp.full_like(m_ref, -1e30)
        l_ref[...] = jnp.zeros_like(l_ref)
        acc_ref[...] = jnp.zeros_like(acc_ref)

    q = q_ref[...]
    k = k_ref[...]
    s = jax.lax.dot_general(q, k, (((1,), (1,)), ((), ())),
                            precision=jax.lax.Precision.HIGHEST,
                            preferred_element_type=jnp.float32) * scale
    cq = cq_ref[...]
    ck = ck_ref[...]
    d2 = jnp.zeros_like(s)
    for t in range(3):
        diff = cq[:, t:t + 1] - ck[:, t:t + 1].T
        d2 = d2 + diff * diff
    s = s - jnp.sqrt(d2 + 1e-8)

    m_prev = m_ref[:, :1]
    m_cur = jnp.maximum(m_prev, jnp.max(s, axis=1, keepdims=True))
    alpha = jnp.exp(m_prev - m_cur)
    p = jnp.exp(s - m_cur)
    l_new = l_ref[:, :1] * alpha + jnp.sum(p, axis=1, keepdims=True)
    acc_new = acc_ref[...] * alpha + jax.lax.dot_general(
        p, v_ref[...], (((1,), (0,)), ((), ())),
        precision=jax.lax.Precision.HIGHEST,
        preferred_element_type=jnp.float32)
    m_ref[...] = jnp.broadcast_to(m_cur, m_ref.shape)
    l_ref[...] = jnp.broadcast_to(l_new, l_ref.shape)
    acc_ref[...] = acc_new

    @pl.when(j == nj - 1)
    def _():
        o_ref[...] = _lrelu(acc_ref[...] / l_new)


def _attn(q, k, v, cq, ck, bm, bn):
    M, d = q.shape
    N = k.shape[0]
    scale = 1.0 / (d ** 0.5)
    grid = (M // bm, N // bn)
    return pl.pallas_call(
        functools.partial(_attn_body, scale=scale),
        grid=grid,
        in_specs=[
            pl.BlockSpec((bm, d), lambda i, j: (i, 0)),
            pl.BlockSpec((bn, d), lambda i, j: (j, 0)),
            pl.BlockSpec((bn, d), lambda i, j: (j, 0)),
            pl.BlockSpec((bm, 8), lambda i, j: (i, 0)),
            pl.BlockSpec((bn, 8), lambda i, j: (j, 0)),
        ],
        out_specs=pl.BlockSpec((bm, d), lambda i, j: (i, 0)),
        out_shape=jax.ShapeDtypeStruct((M, d), jnp.float32),
        scratch_shapes=[
            pltpu.VMEM((bm, d), jnp.float32),
            pltpu.VMEM((bm, 128), jnp.float32),
            pltpu.VMEM((bm, 128), jnp.float32),
        ],
        compiler_params=pltpu.CompilerParams(
            dimension_semantics=("parallel", "arbitrary")),
    )(q, k, v, cq, ck)


def _pad_coords(c):
    return jnp.pad(c, ((0, 0), (0, 5)))


@functools.lru_cache(maxsize=None)
def _sc_seg(NT, NS, D, E, weighted):
    DG = -(-D // 128) * 128
    DR = DG + 128
    CS = min(2048, E)
    KB = 64
    W32 = NS // 32
    fixed = 4 * CS * 4 + (NT * 8 if weighted else 0) + KB * DR * 4 + 4096
    P = 1
    while W32 // P * DR * 4 + fixed > 480_000:
        P *= 2
    WP = W32 // P
    mesh = plsc.VectorSubcoreMesh(core_axis_name="c", subcore_axis_name="s",
                                  num_cores=2, num_subcores=16)

    @functools.partial(
        pl.kernel,
        out_type=jax.ShapeDtypeStruct((NS, DR), jnp.float32),
        mesh=mesh,
        scratch_types=[
            pltpu.VMEM((CS,), jnp.int32),
            pltpu.VMEM((CS,), jnp.int32),
            pltpu.VMEM((NT if weighted else 16,), jnp.float32),
            pltpu.VMEM((NT if weighted else 16,), jnp.float32),
            pltpu.VMEM((CS + 16,), jnp.int32),
            pltpu.VMEM((CS + 16,), jnp.int32),
            pltpu.VMEM((KB, DR), jnp.float32),
            pltpu.VMEM((KB,), jnp.int32),
            pltpu.VMEM((KB,), jnp.int32),
            pltpu.VMEM((KB,), jnp.float32),
            pltpu.VMEM((WP, DR), jnp.float32),
            pltpu.SemaphoreType.DMA,
        ],
        compiler_params=pltpu.CompilerParams(needs_layout_passes=False),
    )
    def k(m_hbm, p1_hbm, p2_hbm, src_hbm, dst_hbm, out_hbm,
          src_st, dst_st, p1_v, p2_v, csrc, cdst, rowsx, srcg, dstg, wg,
          acc, sem):
        c = lax.axis_index("c")
        s = lax.axis_index("s")
        if weighted:
            pltpu.sync_copy(p1_hbm, p1_v)
            pltpu.sync_copy(p2_hbm, p2_v)

        zf = jnp.zeros((16,), jnp.float32)
        zi = jnp.zeros((16,), jnp.int32)
        iot = lax.iota(jnp.int32, 16)

        def zcap(i, _):
            csrc[pl.ds(i * 16, 16)] = zi
            cdst[pl.ds(i * 16, 16)] = zi
            return 0
        lax.fori_loop(0, (CS + 16) // 16, zcap, 0)

        for p in range(P):
            lo = (c * 16 + s) * W32 + p * WP

            def zacc(e, _):
                for f in range(DR // 16):
                    acc[e, pl.ds(f * 16, 16)] = zf
                return 0
            lax.fori_loop(0, WP, zacc, 0)

            def strip(st, _):
                pltpu.sync_copy(src_hbm.at[pl.ds(st * CS, CS)], src_st)
                pltpu.sync_copy(dst_hbm.at[pl.ds(st * CS, CS)], dst_st)

                def scan(i, cnt):
                    sv = src_st[pl.ds(i * 16, 16)]
                    dl = dst_st[pl.ds(i * 16, 16)] - lo
                    mk = (dl >= 0) & (dl < WP)
                    inc = jnp.where(mk, 1, 0).astype(jnp.int32)
                    pos = cnt + plsc.cumsum(inc) - inc
                    plsc.store_scatter(csrc, [pos], sv, mask=mk)
                    plsc.store_scatter(cdst, [pos], dl, mask=mk)
                    pc = plsc.all_reduce_population_count(mk)
                    if getattr(pc, "ndim", 0):
                        pc = jnp.max(pc)
                    return cnt + pc
                cnt = lax.fori_loop(0, CS // 16, scan, jnp.int32(0))

                def chunk(j, _):
                    for t in range(KB // 16):
                        sv = csrc[pl.ds(j * KB + t * 16, 16)]
                        dl = cdst[pl.ds(j * KB + t * 16, 16)]
                        valid = (j * KB + t * 16 + iot) < cnt
                        srcg[pl.ds(t * 16, 16)] = jnp.where(valid, sv, 0)
                        dstg[pl.ds(t * 16, 16)] = dl
                        if weighted:
                            g1 = plsc.load_gather(p1_v, [sv])
                            g2 = plsc.load_gather(p2_v, [dl + lo])
                            e = g1 + g2
                            w = jnp.exp(jnp.where(e >= 0, e, NEG * e))
                        else:
                            w = jnp.ones((16,), jnp.float32)
                        wg[pl.ds(t * 16, 16)] = jnp.where(valid, w, 0.0)
                    pltpu.async_copy(m_hbm.at[srcg], rowsx, sem).wait()

                    def ebody(e2, _):
                        idx = jnp.full((16,), e2, jnp.int32)
                        wsp = plsc.load_gather(wg, [idx])
                        dsp = plsc.load_gather(dstg, [idx])
                        for f in range(DG // 16):
                            plsc.addupdate_scatter(
                                acc, [dsp, f * 16 + iot],
                                rowsx[e2, pl.ds(f * 16, 16)] * wsp)
                        plsc.addupdate_scatter(
                            acc, [dsp, DG + iot],
                            jnp.where(iot == 0, wsp, 0.0))
                        return 0
                    lax.fori_loop(0, KB, ebody, 0)
                    return 0
                lax.fori_loop(0, (cnt + KB - 1) // KB, chunk, 0)
                return 0
            lax.fori_loop(0, E // CS, strip, 0)

            pltpu.sync_copy(acc, out_hbm.at[pl.ds(lo, WP)])

    return k


def _pad128(m):
    d = m.shape[1]
    dr = -(-d // 128) * 128 + 128
    return jnp.pad(m, ((0, 0), (0, dr - d)))


def _gat(h, ei, Ws, As):
    src, dst = ei[0], ei[1]
    n, d = h.shape
    dg = -(-d // 128) * 128
    e = src.shape[0]
    fn = _sc_seg(n, n, d, e, True)
    for l in range(Ws.shape[0]):
        m = h @ Ws[l]
        p1 = m @ As[l][:d]
        p2 = m @ As[l][d:]
        out = fn(_pad128(m), p1, p2, src, dst)
        h = _lrelu(h + out[:, :d] / (out[:, dg:dg + 1] + 1e-9))
    return h


def _seg_sum_sc(x, seg, n_seg):
    nt, d = x.shape
    dg = -(-d // 128) * 128
    fn = _sc_seg(nt, n_seg, d, nt, False)
    dummy = jnp.zeros((16,), jnp.float32)
    iota = jnp.arange(nt, dtype=jnp.int32)
    out = fn(_pad128(x), dummy, dummy, iota, seg)
    return out[:, :d], out[:, dg:dg + 1]


def _atom_interactive(intra, inter, group, Wg, n_groups):
    s, c = _seg_sum_sc(inter, group, n_groups)
    ctx = (s / jnp.maximum(c, 1.0))[group]
    g = jax.nn.sigmoid(jnp.concatenate([intra, inter, ctx], axis=1) @ Wg)
    return intra + g * inter


def _sub_interactive(intra, inter, Wg):
    g = jax.nn.sigmoid(jnp.concatenate([intra, inter], axis=1) @ Wg)
    return _lrelu(inter + g * intra), _lrelu(intra + g * inter)


def _gru_step(x, h, Wi, Wh, bi, bh):
    gi = x @ Wi + bi
    gh = h @ Wh + bh
    r = jax.nn.sigmoid(gi[:FD] + gh[:FD])
    z = jax.nn.sigmoid(gi[FD:2 * FD] + gh[FD:2 * FD])
    n = jnp.tanh(gi[2 * FD:] + r * gh[2 * FD:])
    return (1.0 - z) * n + z * h


def kernel(lig_h, prot_h, lig_coord, prot_coord, frag_x, res_x, frag_coord,
           res_coord, lig_intra_W, lig_intra_a, prot_intra_W, prot_intra_a,
           ia_Wq, ia_Wk, ia_Wvl, ia_Wvp, lig_gate_W, prot_gate_W,
           frag_intra_W, frag_intra_a, res_intra_W, res_intra_a,
           is_Wq, is_Wk, is_Wvl, is_Wvp, lig_sub_gate_W, prot_sub_gate_W,
           gru_Wi, gru_Wh, gru_bi, gru_bh, pred_W, pred_b,
           lig_edge_index, prot_edge_index, frag_edge_index, res_edge_index,
           lig_group, prot_group, atom2frag, atom2res):
    ligand_intra = _gat(lig_h, lig_edge_index, lig_intra_W, lig_intra_a)
    protein_intra = _gat(prot_h, prot_edge_index, prot_intra_W, prot_intra_a)

    ql = ligand_intra @ ia_Wq
    kp = protein_intra @ ia_Wk
    vl = ligand_intra @ ia_Wvl
    vp = protein_intra @ ia_Wvp
    cl = _pad_coords(lig_coord)
    cp = _pad_coords(prot_coord)
    inter_lig = _attn(ql, kp, vp, cl, cp, 256, 1024)
    inter_prot = _attn(kp, ql, vl, cp, cl, 512, 1024)

    H_lig = _atom_interactive(ligand_intra, inter_lig, lig_group,
                              lig_gate_W, NF)
    H_prot = _atom_interactive(protein_intra, inter_prot, prot_group,
                               prot_gate_W, NR)

    frag_h = jnp.concatenate(
        [frag_x, _seg_sum_sc(H_lig, atom2frag, NF)[0]], axis=1)
    res_h = jnp.concatenate(
        [res_x, _seg_sum_sc(H_prot, atom2res, NR)[0]], axis=1)

    lig_sub = _gat(frag_h, frag_edge_index, frag_intra_W, frag_intra_a)
    prot_sub = _gat(res_h, res_edge_index, res_intra_W, res_intra_a)

    qf = lig_sub @ is_Wq
    kr = prot_sub @ is_Wk
    vf = lig_sub @ is_Wvl
    vr = prot_sub @ is_Wvp
    cf = _pad_coords(frag_coord)
    cr = _pad_coords(res_coord)
    inter_l_sub = _attn(qf, kr, vr, cf, cr, 128, 512)
    inter_p_sub = _attn(kr, qf, vf, cr, cf, 128, 128)

    lig_inter_u, lig_intra_u = _sub_interactive(lig_sub, inter_l_sub,
                                                lig_sub_gate_W)
    prot_inter_u, prot_intra_u = _sub_interactive(prot_sub, inter_p_sub,
                                                  prot_sub_gate_W)

    H_final = jnp.concatenate([lig_intra_u.mean(axis=0),
                               prot_intra_u.mean(axis=0)], axis=0)
    Z_final = jnp.concatenate([lig_inter_u.mean(axis=0),
                               prot_inter_u.mean(axis=0)], axis=0)
    h = jnp.zeros((FD,), jnp.float32)
    h = _gru_step(H_final, h, gru_Wi, gru_Wh, gru_bi, gru_bh)
    h = _gru_step(Z_final, h, gru_Wi, gru_Wh, gru_bi, gru_bh)
    return h @ pred_W + pred_b
